# Initial kernel scaffold; baseline (speedup 1.0000x reference)
import jax
import jax.numpy as jnp
from jax.experimental import pallas as pl


def kernel(inputs_raw, gate_w, gate_b, w1, w2):
    raise NotImplementedError("write your pallas kernel here")


# speedup vs baseline: 2.1928x; 2.1928x over previous
import jax
import jax.numpy as jnp
from jax.experimental import pallas as pl
from jax.experimental.pallas import tpu as pltpu

_K = 2


def kernel(inputs_raw, gate_w, gate_b, w1, w2):
    B, T, D = inputs_raw.shape
    E = gate_w.shape[1]
    F = w1.shape[2]
    K = _K

    NT = 8
    Tt = T // NT

    def gate_kernel(x_ref, gw_ref, gb_ref, logits_ref, idx_ref, tw_ref, acc_ref):
        t = pl.program_id(0)

        @pl.when(t == 0)
        def _init():
            acc_ref[...] = jnp.zeros_like(acc_ref)

        acc_ref[...] += jnp.sum(x_ref[...], axis=1)

        @pl.when(t == pl.num_programs(0) - 1)
        def _finish():
            pooled = acc_ref[...] * (1.0 / T)
            logits = jnp.dot(pooled, gw_ref[...],
                             preferred_element_type=jnp.float32) + gb_ref[...]
            logits_ref[...] = logits
            iota = jax.lax.broadcasted_iota(jnp.int32, (B, E), 1)
            v1 = jnp.max(logits, axis=1, keepdims=True)
            i1 = jnp.min(jnp.where(logits == v1, iota, E), axis=1, keepdims=True)
            masked = jnp.where(iota == i1, -jnp.inf, logits)
            v2 = jnp.max(masked, axis=1, keepdims=True)
            i2 = jnp.min(jnp.where(masked == v2, iota, E), axis=1, keepdims=True)
            e2 = jnp.exp(v2 - v1)
            tw1 = 1.0 / (1.0 + e2)
            idx_ref[...] = jnp.concatenate([i1, i2], axis=1).astype(jnp.int32)
            tw_ref[...] = jnp.concatenate([tw1, e2 * tw1], axis=1)

    logits, idx, tw = pl.pallas_call(
        gate_kernel,
        grid=(NT,),
        in_specs=[
            pl.BlockSpec((B, Tt, D), lambda t: (0, t, 0)),
            pl.BlockSpec((D, E), lambda t: (0, 0)),
            pl.BlockSpec((1, E), lambda t: (0, 0)),
        ],
        out_specs=[
            pl.BlockSpec((B, E), lambda t: (0, 0)),
            pl.BlockSpec((B, K), lambda t: (0, 0)),
            pl.BlockSpec((B, K), lambda t: (0, 0)),
        ],
        out_shape=[
            jax.ShapeDtypeStruct((B, E), jnp.float32),
            jax.ShapeDtypeStruct((B, K), jnp.int32),
            jax.ShapeDtypeStruct((B, K), jnp.float32),
        ],
        scratch_shapes=[pltpu.VMEM((B, D), jnp.float32)],
    )(inputs_raw, gate_w, gate_b.reshape(1, E))

    NF = 8
    Ff = F // NF

    def ffn_kernel(idx_ref, tw_ref, x_ref, w1_ref, w2_ref, out_ref):
        b = pl.program_id(0)
        k = pl.program_id(1)
        f = pl.program_id(2)
        x = x_ref[0]
        h = jnp.dot(x.astype(jnp.bfloat16), w1_ref[0].astype(jnp.bfloat16),
                    preferred_element_type=jnp.float32)
        h = jax.nn.gelu(h)
        contrib = jnp.dot(h.astype(jnp.bfloat16), w2_ref[0].astype(jnp.bfloat16),
                          preferred_element_type=jnp.float32)
        tw_val = tw_ref[b, k]
        first = jnp.logical_and(k == 0, f == 0)

        @pl.when(first)
        def _first():
            out_ref[0] = x + tw_val * contrib

        @pl.when(jnp.logical_not(first))
        def _rest():
            out_ref[0] += tw_val * contrib

    grid_spec = pltpu.PrefetchScalarGridSpec(
        num_scalar_prefetch=2,
        grid=(B, K, NF),
        in_specs=[
            pl.BlockSpec((1, T, D), lambda b, k, f, idx, tw: (b, 0, 0)),
            pl.BlockSpec((1, D, Ff), lambda b, k, f, idx, tw: (idx[b, k], 0, f)),
            pl.BlockSpec((1, Ff, D), lambda b, k, f, idx, tw: (idx[b, k], f, 0)),
        ],
        out_specs=pl.BlockSpec((1, T, D), lambda b, k, f, idx, tw: (b, 0, 0)),
    )

    out = pl.pallas_call(
        ffn_kernel,
        grid_spec=grid_spec,
        out_shape=jax.ShapeDtypeStruct((B, T, D), jnp.float32),
        compiler_params=pltpu.CompilerParams(
            dimension_semantics=("parallel", "arbitrary", "arbitrary"),
        ),
    )(idx, tw, inputs_raw, w1, w2)

    return (out, logits)
